# baseline (speedup 1.0000x reference)
import jax
import jax.numpy as jnp
from jax.experimental import pallas as pl
from jax.experimental.pallas import tpu as pltpu

_ROWS = 1024
_NB = 4
_N_TOKENS = 8192
_NCH = _N_TOKENS // _ROWS


def _fused_kernel(x_hbm, w1_h, b1_h, w2_h, b2_h, w3_h, b3_h, o_ref, *scr):
    xbufs = scr[:_NB]
    xsems = scr[_NB:2 * _NB]
    wbufs = scr[2 * _NB:2 * _NB + 6]
    wsems = scr[2 * _NB + 6:2 * _NB + 12]

    w_hbm = (w1_h, b1_h, w2_h, b2_h, w3_h, b3_h)
    wcopies = [pltpu.make_async_copy(h, v, s)
               for h, v, s in zip(w_hbm, wbufs, wsems)]
    for c in wcopies:
        c.start()

    def xcopy(i):
        return pltpu.make_async_copy(
            x_hbm.at[pl.ds(i * _ROWS, _ROWS), :], xbufs[i % _NB],
            xsems[i % _NB])

    for i in range(_NB - 1):
        xcopy(i).start()
    for c in wcopies:
        c.wait()
    w1 = wbufs[0][...].astype(jnp.bfloat16)
    b1 = wbufs[1][...]
    w2 = wbufs[2][...].astype(jnp.bfloat16)
    b2 = wbufs[3][...]
    w3 = wbufs[4][...].astype(jnp.bfloat16)
    b3 = wbufs[5][...]

    for i in range(_NCH):
        if i + _NB - 1 < _NCH:
            xcopy(i + _NB - 1).start()
        xcopy(i).wait()
        x = xbufs[i % _NB][...].astype(jnp.bfloat16)
        h = jnp.dot(x, w1, preferred_element_type=jnp.float32) + b1
        h = jnp.maximum(h, 0.0).astype(jnp.bfloat16)
        h = jnp.dot(h, w2, preferred_element_type=jnp.float32) + b2
        h = jnp.maximum(h, 0.0).astype(jnp.bfloat16)
        o_ref[pl.ds(i * _ROWS, _ROWS), :] = (
            jnp.dot(h, w3, preferred_element_type=jnp.float32) + b3)


def kernel(hidden_states, W1, b1, W2, b2, W3, b3):
    x = hidden_states
    if x.ndim == 3:
        x = jnp.mean(x, axis=1)
    n, d = x.shape
    e = W3.shape[1]
    h1, h2 = W1.shape[1], W2.shape[1]
    return pl.pallas_call(
        _fused_kernel,
        in_specs=[pl.BlockSpec(memory_space=pl.ANY)] * 7,
        out_specs=pl.BlockSpec(memory_space=pltpu.VMEM),
        out_shape=jax.ShapeDtypeStruct((n, e), jnp.float32),
        scratch_shapes=(
            [pltpu.VMEM((_ROWS, d), jnp.float32) for _ in range(_NB)]
            + [pltpu.SemaphoreType.DMA for _ in range(_NB)]
            + [pltpu.VMEM(s, jnp.float32) for s in
               ((d, h1), (1, h1), (h1, h2), (1, h2), (h2, e), (1, e))]
            + [pltpu.SemaphoreType.DMA for _ in range(6)]
        ),
    )(x, W1, b1.reshape(1, -1), W2, b2.reshape(1, -1), W3, b3.reshape(1, -1))
